# baseline (speedup 1.0000x reference)
import functools

import jax
import jax.numpy as jnp
from jax import lax
from jax.experimental import pallas as pl
from jax.experimental.pallas import tpu as pltpu
from jax.experimental.pallas import tpu_sc as plsc

VOCAB = 100000
EMBED = 64
BATCH = 4096
HIST = 50
PADW = 128

_info = plsc.get_sparse_core_info()
NC = _info.num_cores
NS = _info.num_subcores
NW = NC * NS
BCOL = BATCH // NW


@functools.partial(
    pl.kernel,
    mesh=plsc.VectorSubcoreMesh(core_axis_name="c", subcore_axis_name="s"),
    out_type=jax.ShapeDtypeStruct((HIST, EMBED, BATCH), jnp.float32),
    scratch_types=[
        pltpu.VMEM((BCOL,), jnp.int32),
        pltpu.VMEM((BCOL, PADW), jnp.float32),
        pltpu.VMEM((EMBED, BCOL), jnp.float32),
        pltpu.SemaphoreType.DMA,
        pltpu.SemaphoreType.DMA,
    ],
    compiler_params=pltpu.CompilerParams(
        use_tc_tiling_on_sc=True, needs_layout_passes=False),
)
def _lookup(xT_hbm, tpad_hbm, res_hbm, idx_v, rows_v, tile_v, gsem, osem):
    w = lax.axis_index("s") * NC + lax.axis_index("c")
    c0 = w * BCOL
    lane = lax.iota(jnp.int32, 16)
    col = [[((lane + d) & 15) + 16 * t for d in range(16)]
           for t in range(EMBED // 16)]

    def body(h, carry):
        pltpu.sync_copy(xT_hbm.at[h, pl.ds(c0, BCOL)], idx_v)
        pltpu.async_copy(tpad_hbm.at[idx_v], rows_v, gsem).wait()

        @plsc.parallel_loop(0, BCOL // 16)
        def tbody(g):
            rid = lane + 16 * g
            for t in range(EMBED // 16):
                for d in range(16):
                    vals = plsc.load_gather(rows_v, [rid, col[t][d]])
                    plsc.store_scatter(tile_v, [col[t][d], rid], vals)

        pltpu.sync_copy(tile_v, res_hbm.at[h, :, pl.ds(c0, BCOL)])
        return carry

    lax.fori_loop(0, HIST, body, 0)


def kernel(x, table):
    xT = x.T
    tpad = jnp.pad(table, ((0, 0), (0, PADW - EMBED)))
    res = _lookup(xT, tpad)
    return jnp.transpose(res, (2, 0, 1))


# speedup vs baseline: 1.1003x; 1.1003x over previous
import functools

import jax
import jax.numpy as jnp
from jax import lax
from jax.experimental import pallas as pl
from jax.experimental.pallas import tpu as pltpu
from jax.experimental.pallas import tpu_sc as plsc

VOCAB = 100000
EMBED = 64
BATCH = 4096
HIST = 50
PADW = 128

_info = plsc.get_sparse_core_info()
NC = _info.num_cores
NS = _info.num_subcores
NW = NC * NS
BCOL = BATCH // NW


@functools.partial(
    pl.kernel,
    mesh=plsc.VectorSubcoreMesh(core_axis_name="c", subcore_axis_name="s"),
    out_type=jax.ShapeDtypeStruct((HIST, EMBED, BATCH), jnp.float32),
    scratch_types=[
        pltpu.VMEM((2, BCOL), jnp.int32),
        pltpu.VMEM((2, BCOL, PADW), jnp.float32),
        pltpu.VMEM((EMBED, BCOL), jnp.float32),
        pltpu.SemaphoreType.DMA,
        pltpu.SemaphoreType.DMA,
    ],
    compiler_params=pltpu.CompilerParams(
        use_tc_tiling_on_sc=True, needs_layout_passes=False),
)
def _lookup(xT_hbm, tpad_hbm, res_hbm, idx_v, rows_v, tile_v, gsem, osem):
    w = lax.axis_index("s") * NC + lax.axis_index("c")
    c0 = w * BCOL
    lane = lax.iota(jnp.int32, 16)
    col = [[((lane + d) & 15) + 16 * t for d in range(16)]
           for t in range(EMBED // 16)]

    def start_gather(h, buf):
        pltpu.sync_copy(xT_hbm.at[h, pl.ds(c0, BCOL)], idx_v.at[buf])
        pltpu.async_copy(tpad_hbm.at[idx_v.at[buf]], rows_v.at[buf], gsem)

    def wait_gather(buf):
        pltpu.make_async_copy(
            tpad_hbm.at[idx_v.at[buf]], rows_v.at[buf], gsem).wait()

    def transpose_store(h, buf):
        rbuf = rows_v.at[buf]

        @plsc.parallel_loop(0, BCOL // 16)
        def tbody(g):
            rid = lane + 16 * g
            for t in range(EMBED // 16):
                for d in range(16):
                    vals = plsc.load_gather(rbuf, [rid, col[t][d]])
                    plsc.store_scatter(tile_v, [col[t][d], rid], vals)

        pltpu.sync_copy(tile_v, res_hbm.at[h, :, pl.ds(c0, BCOL)])

    start_gather(0, 0)

    def body(hp, carry):
        h0 = 2 * hp
        start_gather(h0 + 1, 1)
        wait_gather(0)
        transpose_store(h0, 0)

        @pl.when(h0 + 2 < HIST)
        def _():
            start_gather(h0 + 2, 0)

        wait_gather(1)
        transpose_store(h0 + 1, 1)
        return carry

    lax.fori_loop(0, HIST // 2, body, 0)


def kernel(x, table):
    xT = x.T
    tpad = jnp.pad(table, ((0, 0), (0, PADW - EMBED)))
    res = _lookup(xT, tpad)
    return jnp.transpose(res, (2, 0, 1))


# speedup vs baseline: 1.2983x; 1.1800x over previous
import functools

import jax
import jax.numpy as jnp
from jax import lax
from jax.experimental import pallas as pl
from jax.experimental.pallas import tpu as pltpu
from jax.experimental.pallas import tpu_sc as plsc

VOCAB = 100000
EMBED = 64
BATCH = 4096
HIST = 50
PADW = 128

_info = plsc.get_sparse_core_info()
NC = _info.num_cores
NS = _info.num_subcores
NW = NC * NS
BCOL = BATCH // NW


@functools.partial(
    pl.kernel,
    mesh=plsc.VectorSubcoreMesh(core_axis_name="c", subcore_axis_name="s"),
    out_type=jax.ShapeDtypeStruct((HIST, EMBED, BATCH), jnp.float32),
    scratch_types=[
        pltpu.VMEM((HIST, BCOL), jnp.int32),
        pltpu.VMEM((2, BCOL, PADW), jnp.float32),
        pltpu.VMEM((EMBED, BCOL), jnp.float32),
        pltpu.SemaphoreType.DMA,
        pltpu.SemaphoreType.DMA,
    ],
    compiler_params=pltpu.CompilerParams(
        use_tc_tiling_on_sc=True, needs_layout_passes=False),
)
def _lookup(xT_hbm, tpad_hbm, res_hbm, idx_v, rows_v, tile_v, gsem, osem):
    w = lax.axis_index("s") * NC + lax.axis_index("c")
    c0 = w * BCOL
    lane = lax.iota(jnp.int32, 16)
    col = [[((lane + d) & 15) + 16 * t for d in range(16)]
           for t in range(EMBED // 16)]

    pltpu.sync_copy(xT_hbm.at[:, pl.ds(c0, BCOL)], idx_v)

    def start_gather(h, buf):
        pltpu.async_copy(tpad_hbm.at[idx_v.at[h]], rows_v.at[buf], gsem)

    def wait_gather(buf):
        pltpu.make_async_copy(
            tpad_hbm.at[idx_v.at[0]], rows_v.at[buf], gsem).wait()

    def transpose_store(h, buf):
        rbuf = rows_v.at[buf]

        @plsc.parallel_loop(0, BCOL // 16)
        def tbody(g):
            rid = lane + 16 * g
            for t in range(EMBED // 16):
                for d in range(16):
                    vals = plsc.load_gather(rbuf, [rid, col[t][d]])
                    plsc.store_scatter(tile_v, [col[t][d], rid], vals)

        pltpu.sync_copy(tile_v, res_hbm.at[h, :, pl.ds(c0, BCOL)])

    start_gather(0, 0)

    def body(hp, carry):
        h0 = 2 * hp
        start_gather(h0 + 1, 1)
        wait_gather(0)
        transpose_store(h0, 0)

        @pl.when(h0 + 2 < HIST)
        def _():
            start_gather(h0 + 2, 0)

        wait_gather(1)
        transpose_store(h0 + 1, 1)
        return carry

    lax.fori_loop(0, HIST // 2, body, 0)


def kernel(x, table):
    xT = x.T
    tpad = jnp.pad(table, ((0, 0), (0, PADW - EMBED)))
    res = _lookup(xT, tpad)
    return jnp.transpose(res, (2, 0, 1))


# speedup vs baseline: 1.3899x; 1.0705x over previous
import functools

import jax
import jax.numpy as jnp
from jax import lax
from jax.experimental import pallas as pl
from jax.experimental.pallas import tpu as pltpu
from jax.experimental.pallas import tpu_sc as plsc

VOCAB = 100000
EMBED = 64
BATCH = 4096
HIST = 50
PADW = 128

_info = plsc.get_sparse_core_info()
NC = _info.num_cores
NS = _info.num_subcores
NW = NC * NS
BCOL = BATCH // NW


@functools.partial(
    pl.kernel,
    mesh=plsc.VectorSubcoreMesh(core_axis_name="c", subcore_axis_name="s"),
    out_type=jax.ShapeDtypeStruct((HIST, EMBED, BATCH), jnp.float32),
    scratch_types=[
        pltpu.VMEM((HIST, BCOL), jnp.int32),
        pltpu.VMEM((2, BCOL, PADW), jnp.float32),
        pltpu.VMEM((2, EMBED, BCOL), jnp.float32),
        pltpu.SemaphoreType.DMA,
        pltpu.SemaphoreType.DMA,
    ],
    compiler_params=pltpu.CompilerParams(
        use_tc_tiling_on_sc=True, needs_layout_passes=False),
)
def _lookup(xT_hbm, tpad_hbm, res_hbm, idx_v, rows_v, tile_v, gsem, osem):
    w = lax.axis_index("s") * NC + lax.axis_index("c")
    c0 = w * BCOL
    lane = lax.iota(jnp.int32, 16)
    col = [[((lane + d) & 15) + 16 * t for d in range(16)]
           for t in range(EMBED // 16)]

    pltpu.sync_copy(xT_hbm.at[:, pl.ds(c0, BCOL)], idx_v)

    def start_gather(h, buf):
        pltpu.async_copy(tpad_hbm.at[idx_v.at[h]], rows_v.at[buf], gsem)

    def wait_gather(buf):
        pltpu.make_async_copy(
            tpad_hbm.at[idx_v.at[0]], rows_v.at[buf], gsem).wait()

    def transpose_store(h, buf):
        rbuf = rows_v.at[buf]
        tbuf = tile_v.at[buf]

        @plsc.parallel_loop(0, BCOL // 16)
        def tbody(g):
            rid = lane + 16 * g
            for t in range(EMBED // 16):
                for d in range(16):
                    vals = plsc.load_gather(rbuf, [rid, col[t][d]])
                    plsc.store_scatter(tbuf, [col[t][d], rid], vals)

        pltpu.async_copy(tbuf, res_hbm.at[h, :, pl.ds(c0, BCOL)], osem)

    def wait_out(buf):
        pltpu.make_async_copy(
            tile_v.at[buf], res_hbm.at[0, :, pl.ds(c0, BCOL)], osem).wait()

    start_gather(0, 0)

    def body(hp, carry):
        h0 = 2 * hp
        start_gather(h0 + 1, 1)
        wait_gather(0)

        @pl.when(hp > 0)
        def _():
            wait_out(0)

        transpose_store(h0, 0)

        @pl.when(h0 + 2 < HIST)
        def _():
            start_gather(h0 + 2, 0)

        wait_gather(1)

        @pl.when(hp > 0)
        def _():
            wait_out(1)

        transpose_store(h0 + 1, 1)
        return carry

    lax.fori_loop(0, HIST // 2, body, 0)
    wait_out(0)
    wait_out(1)


def kernel(x, table):
    xT = x.T
    tpad = jnp.pad(table, ((0, 0), (0, PADW - EMBED)))
    res = _lookup(xT, tpad)
    return jnp.transpose(res, (2, 0, 1))
